# Initial kernel scaffold; baseline (speedup 1.0000x reference)
import functools

import jax
import jax.numpy as jnp
from jax import lax
from jax.experimental import pallas as pl
from jax.experimental.pallas import tpu as pltpu
from jax.experimental.pallas import tpu_sc as plsc

_INFO = plsc.get_sparse_core_info()
_NC = _INFO.num_cores
_NS = _INFO.num_subcores
_NW = _NC * _NS
_CHUNK = 128


@functools.partial(jax.jit, static_argnums=(2, 3))
def _sc_gather(table, idx_flat, n_per_w, n_chunks):
    d = table.shape[1]
    mesh = plsc.VectorSubcoreMesh(core_axis_name="c", subcore_axis_name="s")

    @functools.partial(
        pl.kernel,
        mesh=mesh,
        out_type=jax.ShapeDtypeStruct((n_per_w * _NW, d), jnp.float32),
        scratch_types=[
            pltpu.VMEM((n_per_w,), jnp.int32),
            pltpu.VMEM((_CHUNK, d), jnp.float32),
            pltpu.SemaphoreType.DMA,
        ],
    )
    def k(table_hbm, idx_hbm, out_hbm, idx_v, rows_v, sem):
        wid = lax.axis_index("s") * _NC + lax.axis_index("c")
        base = wid * n_per_w
        pltpu.sync_copy(idx_hbm.at[pl.ds(base, n_per_w)], idx_v)

        def body(j, carry):
            r0 = j * _CHUNK
            cp = pltpu.async_copy(
                table_hbm.at[idx_v.at[pl.ds(r0, _CHUNK)]], rows_v, sem
            )
            cp.wait()
            pltpu.sync_copy(rows_v, out_hbm.at[pl.ds(base + r0, _CHUNK)])
            return carry

        lax.fori_loop(0, n_chunks, body, 0)

    return k(table, idx_flat)


def kernel(src_ids, embed_weight):
    b, l = src_ids.shape
    d = embed_weight.shape[1]
    n = b * l
    idx = src_ids.reshape(n).astype(jnp.int32)
    per_w = _NW * _CHUNK
    n_pad = ((n + per_w - 1) // per_w) * per_w
    if n_pad != n:
        idx = jnp.concatenate([idx, jnp.zeros(n_pad - n, jnp.int32)])
    n_per_w = n_pad // _NW
    out = _sc_gather(embed_weight, idx, n_per_w, n_per_w // _CHUNK)
    return out[:n].reshape(b, l, d)


# speedup vs baseline: 1.6925x; 1.6925x over previous
import functools

import jax
import jax.numpy as jnp
from jax import lax
from jax.experimental import pallas as pl
from jax.experimental.pallas import tpu as pltpu
from jax.experimental.pallas import tpu_sc as plsc

_INFO = plsc.get_sparse_core_info()
_NC = _INFO.num_cores
_NS = _INFO.num_subcores
_NW = _NC * _NS
_CHUNK = 128


@functools.partial(jax.jit, static_argnums=(2, 3))
def _sc_gather(table, idx_flat, n_per_w, n_chunks):
    d = table.shape[1]
    mesh = plsc.VectorSubcoreMesh(core_axis_name="c", subcore_axis_name="s")

    @functools.partial(
        pl.kernel,
        mesh=mesh,
        compiler_params=pltpu.CompilerParams(use_tc_tiling_on_sc=False),
        out_type=jax.ShapeDtypeStruct((n_per_w * _NW, d), jnp.float32),
        scratch_types=[
            pltpu.VMEM((n_per_w,), jnp.int32),
            pltpu.VMEM((_CHUNK, d), jnp.float32),
            pltpu.SemaphoreType.DMA,
        ],
    )
    def k(table_hbm, idx_hbm, out_hbm, idx_v, rows_v, sem):
        wid = lax.axis_index("s") * _NC + lax.axis_index("c")
        base = wid * n_per_w
        pltpu.sync_copy(idx_hbm.at[pl.ds(base, n_per_w)], idx_v)

        def body(j, carry):
            r0 = j * _CHUNK
            cp = pltpu.async_copy(
                table_hbm.at[idx_v.at[pl.ds(r0, _CHUNK)]], rows_v, sem
            )
            cp.wait()
            pltpu.sync_copy(rows_v, out_hbm.at[pl.ds(base + r0, _CHUNK)])
            return carry

        lax.fori_loop(0, n_chunks, body, 0)

    return k(table, idx_flat)


def kernel(src_ids, embed_weight):
    b, l = src_ids.shape
    d = embed_weight.shape[1]
    n = b * l
    idx = src_ids.reshape(n).astype(jnp.int32)
    per_w = _NW * _CHUNK
    n_pad = ((n + per_w - 1) // per_w) * per_w
    if n_pad != n:
        idx = jnp.concatenate([idx, jnp.zeros(n_pad - n, jnp.int32)])
    n_per_w = n_pad // _NW
    out = _sc_gather(embed_weight, idx, n_per_w, n_per_w // _CHUNK)
    return out[:n].reshape(b, l, d)


# speedup vs baseline: 1.8365x; 1.0851x over previous
import functools

import jax
import jax.numpy as jnp
from jax import lax
from jax.experimental import pallas as pl
from jax.experimental.pallas import tpu as pltpu
from jax.experimental.pallas import tpu_sc as plsc

_INFO = plsc.get_sparse_core_info()
_NC = _INFO.num_cores
_NS = _INFO.num_subcores
_NW = _NC * _NS
_CHUNK = 128


@functools.partial(jax.jit, static_argnums=(2, 3))
def _sc_gather(table, idx_flat, n_per_w, n_chunks):
    d = table.shape[1]
    mesh = plsc.VectorSubcoreMesh(core_axis_name="c", subcore_axis_name="s")

    @functools.partial(
        pl.kernel,
        mesh=mesh,
        compiler_params=pltpu.CompilerParams(use_tc_tiling_on_sc=False),
        out_type=jax.ShapeDtypeStruct((n_per_w * _NW, d), jnp.float32),
        scratch_types=[
            pltpu.VMEM((n_per_w,), jnp.int32),
            pltpu.VMEM((_CHUNK, d), jnp.float32),
            pltpu.VMEM((_CHUNK, d), jnp.float32),
            pltpu.SemaphoreType.DMA,
            pltpu.SemaphoreType.DMA,
        ],
    )
    def k(table_hbm, idx_hbm, out_hbm, idx_v, rows0, rows1, sem0, sem1):
        wid = lax.axis_index("s") * _NC + lax.axis_index("c")
        base = wid * n_per_w
        pltpu.sync_copy(idx_hbm.at[pl.ds(base, n_per_w)], idx_v)

        def gather(j, buf, sem):
            return pltpu.async_copy(
                table_hbm.at[idx_v.at[pl.ds(j * _CHUNK, _CHUNK)]], buf, sem
            )

        def store(j, buf):
            pltpu.sync_copy(buf, out_hbm.at[pl.ds(base + j * _CHUNK, _CHUNK)])

        def wait_gather(buf, sem):
            pltpu.make_async_copy(table_hbm.at[pl.ds(0, _CHUNK)], buf, sem).wait()

        gather(0, rows0, sem0)

        def body(i, carry):
            j = i * 2
            gather(j + 1, rows1, sem1)
            wait_gather(rows0, sem0)
            store(j, rows0)
            gather(j + 2, rows0, sem0)
            wait_gather(rows1, sem1)
            store(j + 1, rows1)
            return carry

        lax.fori_loop(0, n_chunks // 2 - 1, body, 0)

        j = n_chunks - 2
        gather(j + 1, rows1, sem1)
        wait_gather(rows0, sem0)
        store(j, rows0)
        wait_gather(rows1, sem1)
        store(j + 1, rows1)

    return k(table, idx_flat)


def kernel(src_ids, embed_weight):
    b, l = src_ids.shape
    d = embed_weight.shape[1]
    n = b * l
    idx = src_ids.reshape(n).astype(jnp.int32)
    per_w = _NW * _CHUNK * 2
    n_pad = ((n + per_w - 1) // per_w) * per_w
    if n_pad != n:
        idx = jnp.concatenate([idx, jnp.zeros(n_pad - n, jnp.int32)])
    n_per_w = n_pad // _NW
    out = _sc_gather(embed_weight, idx, n_per_w, n_per_w // _CHUNK)
    return out[:n].reshape(b, l, d)


# speedup vs baseline: 1.8706x; 1.0186x over previous
import functools

import jax
import jax.numpy as jnp
from jax import lax
from jax.experimental import pallas as pl
from jax.experimental.pallas import tpu as pltpu
from jax.experimental.pallas import tpu_sc as plsc

_INFO = plsc.get_sparse_core_info()
_NC = _INFO.num_cores
_NS = _INFO.num_subcores
_NW = _NC * _NS
_CHUNK = 128
_NBUF = 4


@functools.partial(jax.jit, static_argnums=(2, 3))
def _sc_gather(table, idx_flat, n_per_w, n_groups):
    d = table.shape[1]
    mesh = plsc.VectorSubcoreMesh(core_axis_name="c", subcore_axis_name="s")

    @functools.partial(
        pl.kernel,
        mesh=mesh,
        compiler_params=pltpu.CompilerParams(use_tc_tiling_on_sc=False),
        out_type=jax.ShapeDtypeStruct((n_per_w * _NW, d), jnp.float32),
        scratch_types=(
            [pltpu.VMEM((n_per_w,), jnp.int32)]
            + [pltpu.VMEM((_CHUNK, d), jnp.float32) for _ in range(_NBUF)]
            + [pltpu.SemaphoreType.DMA for _ in range(2 * _NBUF)]
        ),
    )
    def k(table_hbm, idx_hbm, out_hbm, idx_v, *rest):
        bufs = rest[:_NBUF]
        gsems = rest[_NBUF:2 * _NBUF]
        ssems = rest[2 * _NBUF:]
        wid = lax.axis_index("s") * _NC + lax.axis_index("c")
        base = wid * n_per_w
        pltpu.sync_copy(idx_hbm.at[pl.ds(base, n_per_w)], idx_v)

        def gather(j, b):
            pltpu.async_copy(
                table_hbm.at[idx_v.at[pl.ds(j * _CHUNK, _CHUNK)]],
                bufs[b], gsems[b],
            )

        def store(j, b):
            pltpu.async_copy(
                bufs[b], out_hbm.at[pl.ds(base + j * _CHUNK, _CHUNK)], ssems[b]
            )

        def wait_gather(b):
            pltpu.make_async_copy(
                table_hbm.at[pl.ds(0, _CHUNK)], bufs[b], gsems[b]
            ).wait()

        def wait_store(b):
            pltpu.make_async_copy(
                bufs[b], out_hbm.at[pl.ds(base, _CHUNK)], ssems[b]
            ).wait()

        for b in range(_NBUF):
            gather(b, b)

        def body(i, carry):
            j0 = i * _NBUF
            for b in range(_NBUF):
                wait_gather(b)
                store(j0 + b, b)
            for b in range(_NBUF):
                wait_store(b)
                gather(j0 + _NBUF + b, b)
            return carry

        lax.fori_loop(0, n_groups - 1, body, 0)

        j0 = (n_groups - 1) * _NBUF
        for b in range(_NBUF):
            wait_gather(b)
            store(j0 + b, b)
        for b in range(_NBUF):
            wait_store(b)

    return k(table, idx_flat)


def kernel(src_ids, embed_weight):
    b, l = src_ids.shape
    d = embed_weight.shape[1]
    n = b * l
    idx = src_ids.reshape(n).astype(jnp.int32)
    per_w = _NW * _CHUNK * _NBUF
    n_pad = ((n + per_w - 1) // per_w) * per_w
    if n_pad != n:
        idx = jnp.concatenate([idx, jnp.zeros(n_pad - n, jnp.int32)])
    n_per_w = n_pad // _NW
    out = _sc_gather(embed_weight, idx, n_per_w, n_per_w // (_CHUNK * _NBUF))
    return out[:n].reshape(b, l, d)
